# baseline (speedup 1.0000x reference)
import functools

import jax
import jax.numpy as jnp
from jax.experimental import pallas as pl

B, C, H, W = 64, 96, 32, 32
D2 = C // 2
HW = H * W


def _body(x_ref, row_ref, col_ref, out_ref):
    m_row = jax.lax.broadcasted_iota(jnp.int32, (H, HW), 0)
    m_col = jax.lax.broadcasted_iota(jnp.int32, (H, HW), 1)
    sel_tile = (m_col % W == m_row).astype(jnp.float32)
    sel_rep = (m_col // W == m_row).astype(jnp.float32)

    dn = (((0,), (0,)), ((), ()))
    pos_top = jax.lax.dot_general(col_ref[...], sel_tile, dn,
                                  preferred_element_type=jnp.float32)
    pos_bot = jax.lax.dot_general(row_ref[...], sel_rep, dn,
                                  preferred_element_type=jnp.float32)
    pos = jnp.concatenate([pos_top, pos_bot], axis=0)
    out_ref[...] = x_ref[...] + pos[None]


@jax.jit
def kernel(x, row_table, col_table):
    xf = x.reshape(B, C, HW)
    row_e = row_table[:H]
    col_e = col_table[:W]

    bblk = 8
    out = pl.pallas_call(
        _body,
        grid=(B // bblk,),
        in_specs=[
            pl.BlockSpec((bblk, C, HW), lambda i: (i, 0, 0)),
            pl.BlockSpec((H, D2), lambda i: (0, 0)),
            pl.BlockSpec((W, D2), lambda i: (0, 0)),
        ],
        out_specs=pl.BlockSpec((bblk, C, HW), lambda i: (i, 0, 0)),
        out_shape=jax.ShapeDtypeStruct((B, C, HW), jnp.float32),
    )(xf, row_e, col_e)
    return out.reshape(B, C, H, W)
